# baseline (speedup 1.0000x reference)
import functools

import jax
import jax.numpy as jnp
from jax import lax
from jax.experimental import pallas as pl
from jax.experimental.pallas import tpu as pltpu
from jax.experimental.pallas import tpu_sc as plsc

_TOKENS = 32768
_DIM = 768
_E = 8
_EPS = 1e-6
_CAP = float(_TOKENS)
_LANES = 128
_BLK = 1024
_GRID = _TOKENS // _BLK
_NC = 2
_NS = 16
_NW = _NC * _NS
_CHUNK = _TOKENS // _NW
_VL = 16
_GROUPS = _CHUNK // _VL


def _route_block(x_ref, w_ref, b_ref, s_ref, idx_ref, part_ref):
    logits = jnp.dot(
        x_ref[...], w_ref[...],
        preferred_element_type=jnp.float32,
    ) + b_ref[...]
    m = jnp.max(logits, axis=1, keepdims=True)
    z = jnp.sum(jnp.exp(logits - m), axis=1, keepdims=True)
    s = 1.0 / z
    ii = lax.broadcasted_iota(jnp.int32, logits.shape, 1)
    idx = jnp.min(jnp.where(logits >= m, ii, _LANES), axis=1, keepdims=True)
    s_ref[...] = s
    idx_ref[...] = idx
    part_ref[...] = jnp.sum(jnp.where(ii == idx, s, 0.0), axis=0)[None, None, :]


_NSTREAM = 4


def _router_tc(*refs):
    xs = refs[:_NSTREAM]
    w_ref, b_ref = refs[_NSTREAM:_NSTREAM + 2]
    outs = refs[_NSTREAM + 2:]
    for k in range(_NSTREAM):
        _route_block(xs[k], w_ref, b_ref,
                     outs[k], outs[_NSTREAM + k], outs[2 * _NSTREAM + k])


def _x_spec(k):
    return pl.BlockSpec((_BLK, _DIM), lambda i, k=k: (_NSTREAM * i + k, 0))


_tc_call = pl.pallas_call(
    _router_tc,
    grid=(_GRID // _NSTREAM,),
    in_specs=[_x_spec(k) for k in range(_NSTREAM)] + [
        pl.BlockSpec((_DIM, _LANES), lambda i: (0, 0)),
        pl.BlockSpec((1, _LANES), lambda i: (0, 0)),
    ],
    out_specs=(
        [pl.BlockSpec((_BLK, 1), lambda i: (i, 0)) for _ in range(_NSTREAM)]
        + [pl.BlockSpec((_BLK, 1), lambda i: (i, 0)) for _ in range(_NSTREAM)]
        + [pl.BlockSpec((1, 1, _LANES), lambda i: (i, 0, 0))
           for _ in range(_NSTREAM)]
    ),
    out_shape=(
        [jax.ShapeDtypeStruct((_TOKENS // _NSTREAM, 1), jnp.float32)
         for _ in range(_NSTREAM)]
        + [jax.ShapeDtypeStruct((_TOKENS // _NSTREAM, 1), jnp.int32)
           for _ in range(_NSTREAM)]
        + [jax.ShapeDtypeStruct((_GRID // _NSTREAM, 1, _LANES), jnp.float32)
           for _ in range(_NSTREAM)]
    ),
)


def _scatter_sc_body(*refs):
    s_list = refs[:_NSTREAM]
    i_list = refs[_NSTREAM:2 * _NSTREAM]
    p_list = refs[2 * _NSTREAM:3 * _NSTREAM]
    out_hbm = refs[3 * _NSTREAM]
    s_v, i_v, p_v, out_v, g_v = refs[3 * _NSTREAM + 1:]
    wid = lax.axis_index("s") * _NC + lax.axis_index("c")
    base = wid * _CHUNK
    sbase = (wid // _NSTREAM) * _CHUNK
    parity = wid % _NSTREAM
    for k in range(_NSTREAM):
        @pl.when(parity == k)
        def _(k=k):
            pltpu.sync_copy(s_list[k].at[pl.ds(sbase, _CHUNK)], s_v)
            pltpu.sync_copy(i_list[k].at[pl.ds(sbase, _CHUNK)], i_v)

    seg = _GRID * _LANES // _NSTREAM
    for k in range(_NSTREAM):
        pltpu.sync_copy(p_list[k], p_v.at[pl.ds(k * seg, seg)])
    acc = jnp.zeros((_VL,), jnp.float32)
    for r in range(_GRID):
        acc = acc + p_v[pl.ds(r * _LANES, _VL)]
    g_v[...] = _CAP / (acc + _EPS)
    zero = jnp.zeros((_VL,), jnp.float32)
    for k in range(_CHUNK * _E // _VL):
        out_v[pl.ds(k * _VL, _VL)] = zero
    lane = lax.broadcasted_iota(jnp.int32, (_VL,), 0)
    for j in range(_GROUPS):
        sv = s_v[pl.ds(j * _VL, _VL)]
        iv = i_v[pl.ds(j * _VL, _VL)]
        gv = plsc.load_gather(g_v, [iv])
        fidx = (lane + j * _VL) * _E + iv
        plsc.store_scatter(out_v, [fidx], sv * gv)
    pltpu.sync_copy(out_v, out_hbm.at[pl.ds(base * _E, _CHUNK * _E)])


@functools.lru_cache(maxsize=1)
def _get_sc_call():
    return pl.kernel(
        _scatter_sc_body,
        out_type=jax.ShapeDtypeStruct((_TOKENS * _E,), jnp.float32),
        mesh=plsc.VectorSubcoreMesh(
            core_axis_name="c", subcore_axis_name="s",
            num_cores=_NC, num_subcores=_NS,
        ),
        compiler_params=pltpu.CompilerParams(needs_layout_passes=False),
        scratch_types=[
            pltpu.VMEM((_CHUNK,), jnp.float32),
            pltpu.VMEM((_CHUNK,), jnp.int32),
            pltpu.VMEM((_GRID * _LANES,), jnp.float32),
            pltpu.VMEM((_CHUNK * _E,), jnp.float32),
            pltpu.VMEM((_VL,), jnp.float32),
        ],
    )


def kernel(x, W, b):
    w_pad = jnp.zeros((_DIM, _LANES), jnp.float32).at[:, :_E].set(W)
    b_pad = jnp.full((1, _LANES), -1e30, jnp.float32).at[0, :_E].set(b)
    outs = _tc_call(*([x] * _NSTREAM), w_pad, b_pad)
    out = _get_sc_call()(*[o.reshape(-1) for o in outs])
    return out.reshape(_TOKENS, _E)


# speedup vs baseline: 1.0584x; 1.0584x over previous
import functools

import jax
import jax.numpy as jnp
from jax import lax
from jax.experimental import pallas as pl
from jax.experimental.pallas import tpu as pltpu
from jax.experimental.pallas import tpu_sc as plsc

_TOKENS = 32768
_DIM = 768
_E = 8
_EPS = 1e-6
_CAP = float(_TOKENS)
_LANES = 128
_BLK = 2048
_GRID = _TOKENS // _BLK
_NC = 2
_NS = 16
_NW = _NC * _NS
_CHUNK = _TOKENS // _NW
_VL = 16
_GROUPS = _CHUNK // _VL


def _route_block(x_ref, w_ref, b_ref, s_ref, idx_ref, part_ref):
    logits = jnp.dot(
        x_ref[...], w_ref[...],
        preferred_element_type=jnp.float32,
    ) + b_ref[...]
    m = jnp.max(logits, axis=1, keepdims=True)
    z = jnp.sum(jnp.exp(logits - m), axis=1, keepdims=True)
    s = 1.0 / z
    ii = lax.broadcasted_iota(jnp.int32, (1, _LANES), 1).astype(jnp.float32)
    idx = jnp.min(jnp.where(logits >= m, ii, float(_LANES)), axis=1,
                  keepdims=True)
    s_ref[...] = s
    idx_ref[...] = idx.astype(jnp.int32)
    part_ref[...] = jnp.sum(jnp.where(ii == idx, s, 0.0), axis=0)[None, None, :]


_NSTREAM = 2


def _router_tc(*refs):
    xs = refs[:_NSTREAM]
    w_ref, b_ref = refs[_NSTREAM:_NSTREAM + 2]
    outs = refs[_NSTREAM + 2:]
    for k in range(_NSTREAM):
        _route_block(xs[k], w_ref, b_ref,
                     outs[k], outs[_NSTREAM + k], outs[2 * _NSTREAM + k])


def _x_spec(k):
    return pl.BlockSpec((_BLK, _DIM), lambda i, k=k: (_NSTREAM * i + k, 0))


_tc_call = pl.pallas_call(
    _router_tc,
    grid=(_GRID // _NSTREAM,),
    in_specs=[_x_spec(k) for k in range(_NSTREAM)] + [
        pl.BlockSpec((_DIM, _LANES), lambda i: (0, 0)),
        pl.BlockSpec((1, _LANES), lambda i: (0, 0)),
    ],
    out_specs=(
        [pl.BlockSpec((_BLK, 1), lambda i: (i, 0)) for _ in range(_NSTREAM)]
        + [pl.BlockSpec((_BLK, 1), lambda i: (i, 0)) for _ in range(_NSTREAM)]
        + [pl.BlockSpec((1, 1, _LANES), lambda i: (i, 0, 0))
           for _ in range(_NSTREAM)]
    ),
    out_shape=(
        [jax.ShapeDtypeStruct((_TOKENS // _NSTREAM, 1), jnp.float32)
         for _ in range(_NSTREAM)]
        + [jax.ShapeDtypeStruct((_TOKENS // _NSTREAM, 1), jnp.int32)
           for _ in range(_NSTREAM)]
        + [jax.ShapeDtypeStruct((_GRID // _NSTREAM, 1, _LANES), jnp.float32)
           for _ in range(_NSTREAM)]
    ),
)


def _scatter_sc_body(*refs):
    s_list = refs[:_NSTREAM]
    i_list = refs[_NSTREAM:2 * _NSTREAM]
    p_list = refs[2 * _NSTREAM:3 * _NSTREAM]
    out_hbm = refs[3 * _NSTREAM]
    s_v, i_v, p_v, out_v, g_v = refs[3 * _NSTREAM + 1:]
    wid = lax.axis_index("s") * _NC + lax.axis_index("c")
    base = wid * _CHUNK
    tcb = base // _BLK
    parity = tcb % _NSTREAM
    sbase = (tcb // _NSTREAM) * _BLK + base % _BLK
    for k in range(_NSTREAM):
        @pl.when(parity == k)
        def _(k=k):
            pltpu.sync_copy(s_list[k].at[pl.ds(sbase, _CHUNK)], s_v)
            pltpu.sync_copy(i_list[k].at[pl.ds(sbase, _CHUNK)], i_v)

    seg = _GRID * _LANES // _NSTREAM
    for k in range(_NSTREAM):
        pltpu.sync_copy(p_list[k], p_v.at[pl.ds(k * seg, seg)])
    acc = jnp.zeros((_VL,), jnp.float32)
    for r in range(_GRID):
        acc = acc + p_v[pl.ds(r * _LANES, _VL)]
    g_v[...] = _CAP / (acc + _EPS)
    zero = jnp.zeros((_VL,), jnp.float32)
    for k in range(_CHUNK * _E // _VL):
        out_v[pl.ds(k * _VL, _VL)] = zero
    lane = lax.broadcasted_iota(jnp.int32, (_VL,), 0)
    for j in range(_GROUPS):
        sv = s_v[pl.ds(j * _VL, _VL)]
        iv = i_v[pl.ds(j * _VL, _VL)]
        gv = plsc.load_gather(g_v, [iv])
        fidx = (lane + j * _VL) * _E + iv
        plsc.store_scatter(out_v, [fidx], sv * gv)
    pltpu.sync_copy(out_v, out_hbm.at[pl.ds(base * _E, _CHUNK * _E)])


@functools.lru_cache(maxsize=1)
def _get_sc_call():
    return pl.kernel(
        _scatter_sc_body,
        out_type=jax.ShapeDtypeStruct((_TOKENS * _E,), jnp.float32),
        mesh=plsc.VectorSubcoreMesh(
            core_axis_name="c", subcore_axis_name="s",
            num_cores=_NC, num_subcores=_NS,
        ),
        compiler_params=pltpu.CompilerParams(needs_layout_passes=False),
        scratch_types=[
            pltpu.VMEM((_CHUNK,), jnp.float32),
            pltpu.VMEM((_CHUNK,), jnp.int32),
            pltpu.VMEM((_GRID * _LANES,), jnp.float32),
            pltpu.VMEM((_CHUNK * _E,), jnp.float32),
            pltpu.VMEM((_VL,), jnp.float32),
        ],
    )


def kernel(x, W, b):
    w_pad = jnp.zeros((_DIM, _LANES), jnp.float32).at[:, :_E].set(W)
    b_pad = jnp.full((1, _LANES), -1e30, jnp.float32).at[0, :_E].set(b)
    outs = _tc_call(*([x] * _NSTREAM), w_pad, b_pad)
    out = _get_sc_call()(*[o.reshape(-1) for o in outs])
    return out.reshape(_TOKENS, _E)
